# baseline (speedup 1.0000x reference)
import functools

import jax
import jax.numpy as jnp
from jax import lax
from jax.experimental import pallas as pl
from jax.experimental.pallas import tpu as pltpu
from jax.experimental.pallas import tpu_sc as plsc

_B = 4096
_D = 16
_C = 16
_K = 15

_NC = 2
_NS = 16
_NW = _NC * _NS
_RW = _B // _NW
_CH = 8
_NCHUNK = _RW // _CH
_NV = _B // 16
_GS = 8
_NG = _NV // _GS


def _dist_body(e_ref, et_ref, o_ref):
    e = e_ref[...]
    et = et_ref[...]
    x2i = jnp.sum(e * e, axis=1, keepdims=True)
    x2j = jnp.sum(et * et, axis=0, keepdims=True)
    d2 = x2i + x2j - 2.0 * jnp.dot(e, et, preferred_element_type=jnp.float32)
    o_ref[...] = jnp.sqrt(jnp.maximum(d2, 0.0))


def _dist_matrix(encodings, encodings_t):
    rb = 256
    return pl.pallas_call(
        _dist_body,
        grid=(_B // rb,),
        in_specs=[
            pl.BlockSpec((rb, _D), lambda i: (i, 0)),
            pl.BlockSpec((_D, _B), lambda i: (0, 0)),
        ],
        out_specs=pl.BlockSpec((rb, _B), lambda i: (i, 0)),
        out_shape=jax.ShapeDtypeStruct((_B, _B), jnp.float32),
    )(encodings, encodings_t)


def _labels_body(cat_ref, lab_ref, e0_ref, e1_ref, gent_ref, npop_ref):
    cat = cat_ref[...]
    mx = jnp.max(cat, axis=1, keepdims=True)
    iota = lax.broadcasted_iota(jnp.int32, (_B, _C), 1)
    ismax = cat == mx
    lab = jnp.min(jnp.where(ismax, iota, _C), axis=1, keepdims=True)
    lab_ref[...] = lab
    one = jnp.ones_like(lab)
    sh0 = 4 * jnp.minimum(lab, 7)
    sh1 = 4 * jnp.clip(lab - 8, 0, 7)
    e0_ref[...] = jnp.where(lab < 8, one << sh0, 0)
    e1_ref[...] = jnp.where(lab >= 8, one << sh1, 0)
    onehot = (iota == lab).astype(jnp.float32)
    g = jnp.sum(onehot, axis=0)
    gb = g * jnp.float32(1.0 / _B)
    gent_ref[...] = (-jnp.sum(gb * jnp.log(gb + 1e-5)))[None, None]
    npop_ref[...] = jnp.sum((g > 0).astype(jnp.float32))[None, None]


def _labels_call(categorical):
    return pl.pallas_call(
        _labels_body,
        out_shape=(
            jax.ShapeDtypeStruct((_B, 1), jnp.int32),
            jax.ShapeDtypeStruct((_B, 1), jnp.int32),
            jax.ShapeDtypeStruct((_B, 1), jnp.int32),
            jax.ShapeDtypeStruct((1, 1), jnp.float32),
            jax.ShapeDtypeStruct((1, 1), jnp.float32),
        ),
    )(categorical)


def _sc_counts(dist_flat, enc0, enc1):
    mesh = plsc.VectorSubcoreMesh(core_axis_name="c", subcore_axis_name="s")

    @functools.partial(
        pl.kernel,
        mesh=mesh,
        compiler_params=pltpu.CompilerParams(needs_layout_passes=False),
        out_type=jax.ShapeDtypeStruct((_B * _C,), jnp.float32),
        scratch_types=[
            pltpu.VMEM((_CH * _B,), jnp.float32),
            pltpu.VMEM((_B,), jnp.int32),
            pltpu.VMEM((_B,), jnp.int32),
            pltpu.VMEM((_RW * _C,), jnp.float32),
            pltpu.VMEM((_NG * 16,), jnp.float32),
        ],
    )
    def body(dist_hbm, enc0_hbm, enc1_hbm, out_hbm, row_v, e0_v, e1_v, out_v,
             gm_v):
        wid = lax.axis_index("s") * _NC + lax.axis_index("c")
        base = wid * _RW
        pltpu.sync_copy(enc0_hbm, e0_v)
        pltpu.sync_copy(enc1_hbm, e1_v)

        def chunk_body(c, _):
            start = (base + c * _CH) * _B
            pltpu.sync_copy(dist_hbm.at[pl.ds(start, _CH * _B)], row_v)

            def row_body(r, _r):
                roff = r * _B

                def pre(g, _g):
                    b = roff + g * (16 * _GS)
                    cs = [row_v[pl.ds(b + j * 16, 16)] for j in range(_GS)]
                    while len(cs) > 1:
                        cs = [jnp.minimum(cs[k], cs[k + 1])
                              for k in range(0, len(cs) - 1, 2)] + (
                                  [cs[-1]] if len(cs) % 2 else [])
                    gm_v[pl.ds(g * 16, 16)] = cs[0]
                    return 0

                lax.fori_loop(0, _NG, pre, 0)

                def p1(g, carry):
                    top0, mt0 = carry
                    gm = gm_v[pl.ds(g * 16, 16)]

                    def active(carry_a):
                        top, mt = carry_a
                        b = roff + g * (16 * _GS)
                        for j in range(_GS):
                            cv = row_v[pl.ds(b + j * 16, 16)]

                            def merge(carry_m):
                                tc, _mc = carry_m
                                cs = lax.sort(cv)
                                ts = lax.sort(
                                    jnp.minimum(tc, lax.rev(cs, (0,))))
                                return ts, jnp.max(ts)

                            top, mt = lax.cond(
                                jnp.any(cv < mt), merge,
                                lambda carry_m: carry_m, (top, mt))
                        return top, mt

                    return lax.cond(jnp.any(gm < mt0), active,
                                    lambda carry_a: carry_a, (top0, mt0))

                inf16 = jnp.full((16,), jnp.inf, jnp.float32)
                _top, t = lax.fori_loop(0, _NG, p1, (inf16, jnp.inf))

                def p2(g, carry):
                    a00, a10 = carry
                    gm = gm_v[pl.ds(g * 16, 16)]

                    def active(carry_a):
                        a0, a1 = carry_a
                        b = roff + g * (16 * _GS)
                        eb = g * (16 * _GS)
                        z = jnp.zeros((16,), jnp.int32)
                        for j in range(_GS):
                            cv = row_v[pl.ds(b + j * 16, 16)]
                            m = cv < t
                            a0 = a0 + jnp.where(m, e0_v[pl.ds(eb + j * 16, 16)], z)
                            a1 = a1 + jnp.where(m, e1_v[pl.ds(eb + j * 16, 16)], z)
                        return a0, a1

                    return lax.cond(jnp.any(gm < t), active,
                                    lambda carry_a: carry_a, (a00, a10))

                z16 = jnp.zeros((16,), jnp.int32)
                a0, a1 = lax.fori_loop(0, _NG, p2, (z16, z16))
                s0 = jnp.sum(a0)
                s1 = jnp.sum(a1)
                lane = lax.iota(jnp.int32, 16)
                sh = 4 * (lane & 7)
                c0 = (s0 >> sh) & 15
                c1 = (s1 >> sh) & 15
                cv16 = jnp.where(lane < 8, c0, c1).astype(jnp.float32)
                out_v[pl.ds((c * _CH + r) * _C, _C)] = cv16
                return 0

            lax.fori_loop(0, _CH, row_body, 0)
            return 0

        lax.fori_loop(0, _NCHUNK, chunk_body, 0)
        pltpu.sync_copy(out_v, out_hbm.at[pl.ds(base * _C, _RW * _C)])

    return body(dist_flat, enc0, enc1)


def _entropy_body(cnt_ref, nent_ref):
    cnt = cnt_ref[...]
    ns = jnp.sum(cnt, axis=1, keepdims=True)
    bins = cnt / ns
    nent_ref[...] = -jnp.sum(bins * jnp.log(bins + 1e-5), axis=1, keepdims=True)


def _entropy_call(counts):
    return pl.pallas_call(
        _entropy_body,
        out_shape=jax.ShapeDtypeStruct((_B, 1), jnp.float32),
    )(counts)


def kernel(encodings, categorical):
    dist = _dist_matrix(encodings, encodings.T)
    lab, enc0, enc1, gent, npop = _labels_call(categorical)
    del lab
    counts_flat = _sc_counts(
        dist.reshape(_B * _B),
        enc0.reshape(_B),
        enc1.reshape(_B),
    )
    nent = _entropy_call(counts_flat.reshape(_B, _C))
    return (
        encodings,
        nent.reshape(_B),
        gent.reshape(()),
        npop.reshape(()),
    )


# speedup vs baseline: 1.1730x; 1.1730x over previous
import functools

import jax
import jax.numpy as jnp
from jax import lax
from jax.experimental import pallas as pl
from jax.experimental.pallas import tpu as pltpu
from jax.experimental.pallas import tpu_sc as plsc

_B = 4096
_D = 16
_C = 16
_K = 15

_NC = 2
_NS = 16
_NW = _NC * _NS
_RW = _B // _NW
_CH = 8
_NCHUNK = _RW // _CH
_NV = _B // 16
_GS = 4
_NG = _NV // _GS
_Q = _B // _GS


def _dist_body(e_ref, et_ref, o_ref, gm_ref):
    e = e_ref[...]
    et = et_ref[...]
    x2i = jnp.sum(e * e, axis=1, keepdims=True)
    x2j = jnp.sum(et * et, axis=0, keepdims=True)
    d2 = x2i + x2j - 2.0 * jnp.dot(e, et, preferred_element_type=jnp.float32)
    d = jnp.sqrt(jnp.maximum(d2, 0.0))
    o_ref[...] = d
    gm_ref[...] = jnp.minimum(
        jnp.minimum(d[:, 0:_Q], d[:, _Q:2 * _Q]),
        jnp.minimum(d[:, 2 * _Q:3 * _Q], d[:, 3 * _Q:4 * _Q]))


def _dist_matrix(encodings, encodings_t):
    rb = 256
    return pl.pallas_call(
        _dist_body,
        grid=(_B // rb,),
        in_specs=[
            pl.BlockSpec((rb, _D), lambda i: (i, 0)),
            pl.BlockSpec((_D, _B), lambda i: (0, 0)),
        ],
        out_specs=[
            pl.BlockSpec((rb, _B), lambda i: (i, 0)),
            pl.BlockSpec((rb, _Q), lambda i: (i, 0)),
        ],
        out_shape=(
            jax.ShapeDtypeStruct((_B, _B), jnp.float32),
            jax.ShapeDtypeStruct((_B, _Q), jnp.float32),
        ),
    )(encodings, encodings_t)


def _labels_body(cat_ref, lab_ref, e0_ref, e1_ref, gent_ref, npop_ref):
    cat = cat_ref[...]
    mx = jnp.max(cat, axis=1, keepdims=True)
    iota = lax.broadcasted_iota(jnp.int32, (_B, _C), 1)
    ismax = cat == mx
    lab = jnp.min(jnp.where(ismax, iota, _C), axis=1, keepdims=True)
    lab_ref[...] = lab
    one = jnp.ones_like(lab)
    sh0 = 4 * jnp.minimum(lab, 7)
    sh1 = 4 * jnp.clip(lab - 8, 0, 7)
    e0_ref[...] = jnp.where(lab < 8, one << sh0, 0)
    e1_ref[...] = jnp.where(lab >= 8, one << sh1, 0)
    onehot = (iota == lab).astype(jnp.float32)
    g = jnp.sum(onehot, axis=0)
    gb = g * jnp.float32(1.0 / _B)
    gent_ref[...] = (-jnp.sum(gb * jnp.log(gb + 1e-5)))[None, None]
    npop_ref[...] = jnp.sum((g > 0).astype(jnp.float32))[None, None]


def _labels_call(categorical):
    return pl.pallas_call(
        _labels_body,
        out_shape=(
            jax.ShapeDtypeStruct((_B, 1), jnp.int32),
            jax.ShapeDtypeStruct((_B, 1), jnp.int32),
            jax.ShapeDtypeStruct((_B, 1), jnp.int32),
            jax.ShapeDtypeStruct((1, 1), jnp.float32),
            jax.ShapeDtypeStruct((1, 1), jnp.float32),
        ),
    )(categorical)


def _sc_counts(dist_flat, gm_flat, enc0, enc1):
    mesh = plsc.VectorSubcoreMesh(core_axis_name="c", subcore_axis_name="s")

    @functools.partial(
        pl.kernel,
        mesh=mesh,
        compiler_params=pltpu.CompilerParams(needs_layout_passes=False),
        out_type=jax.ShapeDtypeStruct((_B * _C,), jnp.float32),
        scratch_types=[
            pltpu.VMEM((_CH * _B,), jnp.float32),
            pltpu.VMEM((_CH * _Q,), jnp.float32),
            pltpu.VMEM((_B,), jnp.int32),
            pltpu.VMEM((_B,), jnp.int32),
            pltpu.VMEM((_RW * _C,), jnp.float32),
        ],
    )
    def body(dist_hbm, gm_hbm, enc0_hbm, enc1_hbm, out_hbm, row_v, gm_v,
             e0_v, e1_v, out_v):
        wid = lax.axis_index("s") * _NC + lax.axis_index("c")
        base = wid * _RW
        pltpu.sync_copy(enc0_hbm, e0_v)
        pltpu.sync_copy(enc1_hbm, e1_v)

        def chunk_body(c, _):
            start = base + c * _CH
            pltpu.sync_copy(dist_hbm.at[pl.ds(start * _B, _CH * _B)], row_v)
            pltpu.sync_copy(gm_hbm.at[pl.ds(start * _Q, _CH * _Q)], gm_v)

            def row_body(r, _r):
                roff = r * _B
                goff = r * _Q


                def p1(g, carry):
                    top0, mt0 = carry
                    gm = gm_v[pl.ds(goff + g * 16, 16)]

                    def active(carry_a):
                        top, mt = carry_a
                        b = roff + g * 16
                        for j in range(_GS):
                            cv = row_v[pl.ds(b + j * _Q, 16)]

                            def merge(carry_m):
                                tc, _mc = carry_m
                                cs = lax.sort(cv)
                                ts = lax.sort(
                                    jnp.minimum(tc, lax.rev(cs, (0,))))
                                return ts, jnp.max(ts)

                            top, mt = lax.cond(
                                jnp.any(cv < mt), merge,
                                lambda carry_m: carry_m, (top, mt))
                        return top, mt

                    return lax.cond(jnp.any(gm < mt0), active,
                                    lambda carry_a: carry_a, (top0, mt0))

                inf16 = jnp.full((16,), jnp.inf, jnp.float32)
                _top, t = lax.fori_loop(0, _NG, p1, (inf16, jnp.inf))

                def p2(g, carry):
                    a00, a10 = carry
                    gm = gm_v[pl.ds(goff + g * 16, 16)]

                    def active(carry_a):
                        a0, a1 = carry_a
                        b = roff + g * 16
                        eb = g * 16
                        z = jnp.zeros((16,), jnp.int32)
                        for j in range(_GS):
                            cv = row_v[pl.ds(b + j * _Q, 16)]
                            m = cv < t
                            a0 = a0 + jnp.where(m, e0_v[pl.ds(eb + j * _Q, 16)], z)
                            a1 = a1 + jnp.where(m, e1_v[pl.ds(eb + j * _Q, 16)], z)
                        return a0, a1

                    return lax.cond(jnp.any(gm < t), active,
                                    lambda carry_a: carry_a, (a00, a10))

                z16 = jnp.zeros((16,), jnp.int32)
                a0, a1 = lax.fori_loop(0, _NG, p2, (z16, z16))
                s0 = jnp.sum(a0)
                s1 = jnp.sum(a1)
                lane = lax.iota(jnp.int32, 16)
                sh = 4 * (lane & 7)
                c0 = (s0 >> sh) & 15
                c1 = (s1 >> sh) & 15
                cv16 = jnp.where(lane < 8, c0, c1).astype(jnp.float32)
                out_v[pl.ds((c * _CH + r) * _C, _C)] = cv16
                return 0

            lax.fori_loop(0, _CH, row_body, 0)
            return 0

        lax.fori_loop(0, _NCHUNK, chunk_body, 0)
        pltpu.sync_copy(out_v, out_hbm.at[pl.ds(base * _C, _RW * _C)])

    return body(dist_flat, gm_flat, enc0, enc1)


def _entropy_body(cnt_ref, nent_ref):
    cnt = cnt_ref[...]
    ns = jnp.sum(cnt, axis=1, keepdims=True)
    bins = cnt / ns
    nent_ref[...] = -jnp.sum(bins * jnp.log(bins + 1e-5), axis=1, keepdims=True)


def _entropy_call(counts):
    return pl.pallas_call(
        _entropy_body,
        out_shape=jax.ShapeDtypeStruct((_B, 1), jnp.float32),
    )(counts)


def kernel(encodings, categorical):
    dist, gmat = _dist_matrix(encodings, encodings.T)
    lab, enc0, enc1, gent, npop = _labels_call(categorical)
    del lab
    counts_flat = _sc_counts(
        dist.reshape(_B * _B),
        gmat.reshape(_B * _Q),
        enc0.reshape(_B),
        enc1.reshape(_B),
    )
    nent = _entropy_call(counts_flat.reshape(_B, _C))
    return (
        encodings,
        nent.reshape(_B),
        gent.reshape(()),
        npop.reshape(()),
    )


# speedup vs baseline: 1.4268x; 1.2163x over previous
import functools

import jax
import jax.numpy as jnp
from jax import lax
from jax.experimental import pallas as pl
from jax.experimental.pallas import tpu as pltpu
from jax.experimental.pallas import tpu_sc as plsc

_B = 4096
_D = 16
_C = 16
_K = 15

_NC = 2
_NS = 16
_NW = _NC * _NS
_RW = _B // _NW
_CH = 8
_NCHUNK = _RW // _CH
_GS = 4
_NG = _B // (16 * _GS)
_SG = _NG // 16
_Q = _B // _GS


def _dist_body(e_ref, et_ref, o_ref, gm_ref):
    e = e_ref[...]
    et = et_ref[...]
    x2i = jnp.sum(e * e, axis=1, keepdims=True)
    x2j = jnp.sum(et * et, axis=0, keepdims=True)
    d2 = x2i + x2j - 2.0 * jnp.dot(e, et, preferred_element_type=jnp.float32)
    d = jnp.sqrt(jnp.maximum(d2, 0.0))
    o_ref[...] = d
    q = jnp.minimum(
        jnp.minimum(d[:, 0:_Q], d[:, _Q:2 * _Q]),
        jnp.minimum(d[:, 2 * _Q:3 * _Q], d[:, 3 * _Q:4 * _Q]))
    gm_ref[...] = jnp.min(q.reshape(q.shape[0], _NG, 16), axis=2)


def _dist_matrix(encodings, encodings_t):
    rb = 256
    return pl.pallas_call(
        _dist_body,
        grid=(_B // rb,),
        in_specs=[
            pl.BlockSpec((rb, _D), lambda i: (i, 0)),
            pl.BlockSpec((_D, _B), lambda i: (0, 0)),
        ],
        out_specs=[
            pl.BlockSpec((rb, _B), lambda i: (i, 0)),
            pl.BlockSpec((rb, _NG), lambda i: (i, 0)),
        ],
        out_shape=(
            jax.ShapeDtypeStruct((_B, _B), jnp.float32),
            jax.ShapeDtypeStruct((_B, _NG), jnp.float32),
        ),
    )(encodings, encodings_t)


def _labels_body(cat_ref, lab_ref, e0_ref, e1_ref, gent_ref, npop_ref):
    cat = cat_ref[...]
    mx = jnp.max(cat, axis=1, keepdims=True)
    iota = lax.broadcasted_iota(jnp.int32, (_B, _C), 1)
    ismax = cat == mx
    lab = jnp.min(jnp.where(ismax, iota, _C), axis=1, keepdims=True)
    lab_ref[...] = lab
    one = jnp.ones_like(lab)
    sh0 = 4 * jnp.minimum(lab, 7)
    sh1 = 4 * jnp.clip(lab - 8, 0, 7)
    e0_ref[...] = jnp.where(lab < 8, one << sh0, 0)
    e1_ref[...] = jnp.where(lab >= 8, one << sh1, 0)
    onehot = (iota == lab).astype(jnp.float32)
    g = jnp.sum(onehot, axis=0)
    gb = g * jnp.float32(1.0 / _B)
    gent_ref[...] = (-jnp.sum(gb * jnp.log(gb + 1e-5)))[None, None]
    npop_ref[...] = jnp.sum((g > 0).astype(jnp.float32))[None, None]


def _labels_call(categorical):
    return pl.pallas_call(
        _labels_body,
        out_shape=(
            jax.ShapeDtypeStruct((_B, 1), jnp.int32),
            jax.ShapeDtypeStruct((_B, 1), jnp.int32),
            jax.ShapeDtypeStruct((_B, 1), jnp.int32),
            jax.ShapeDtypeStruct((1, 1), jnp.float32),
            jax.ShapeDtypeStruct((1, 1), jnp.float32),
        ),
    )(categorical)


def _sc_counts(dist_flat, gm_flat, enc0, enc1):
    mesh = plsc.VectorSubcoreMesh(core_axis_name="c", subcore_axis_name="s")

    @functools.partial(
        pl.kernel,
        mesh=mesh,
        compiler_params=pltpu.CompilerParams(needs_layout_passes=False),
        out_type=jax.ShapeDtypeStruct((_B * _C,), jnp.float32),
        scratch_types=[
            pltpu.VMEM((_CH * _B,), jnp.float32),
            pltpu.VMEM((_CH * _NG,), jnp.float32),
            pltpu.VMEM((_B,), jnp.int32),
            pltpu.VMEM((_B,), jnp.int32),
            pltpu.VMEM((_RW * _C,), jnp.float32),
        ],
    )
    def body(dist_hbm, gm_hbm, enc0_hbm, enc1_hbm, out_hbm, row_v, gm_v,
             e0_v, e1_v, out_v):
        wid = lax.axis_index("s") * _NC + lax.axis_index("c")
        base = wid * _RW
        pltpu.sync_copy(enc0_hbm, e0_v)
        pltpu.sync_copy(enc1_hbm, e1_v)
        iota16 = lax.iota(jnp.int32, 16)

        def chunk_body(c, _):
            start = base + c * _CH
            pltpu.sync_copy(dist_hbm.at[pl.ds(start * _B, _CH * _B)], row_v)
            pltpu.sync_copy(gm_hbm.at[pl.ds(start * _NG, _CH * _NG)], gm_v)

            def row_body(r, _r):
                roff = r * _B
                goff = r * _NG

                gsv = [gm_v[pl.ds(goff + s * 16, 16)] for s in range(_SG)]

                s0 = lax.sort(gsv[0])
                s1 = lax.sort(gsv[1])
                s2 = lax.sort(gsv[2])
                s3 = lax.sort(gsv[3])
                m01 = lax.sort(jnp.minimum(s0, lax.rev(s1, (0,))))
                m23 = lax.sort(jnp.minimum(s2, lax.rev(s3, (0,))))
                mm = lax.sort(jnp.minimum(m01, lax.rev(m23, (0,))))
                t0 = jnp.max(mm)

                top = jnp.full((16,), jnp.inf, jnp.float32)
                mt = t0
                for s in range(_SG):
                    sg = gsv[s]
                    sb = roff + s * 256

                    def p1_cond(carry):
                        mk, _t, _m = carry
                        return jnp.any(mk)

                    def p1_body(carry, sg=sg, sb=sb):
                        mk, tp, m_ = carry
                        g = jnp.max(plsc.all_reduce_ffs(mk))
                        b = sb + g * 16
                        for j in range(_GS):
                            cv = row_v[pl.ds(b + j * _Q, 16)]

                            def merge(cm, cv=cv):
                                tc, _mc = cm
                                ts = lax.sort(jnp.minimum(
                                    tc, lax.rev(lax.sort(cv), (0,))))
                                return ts, jnp.max(ts)

                            tp, m_ = lax.cond(
                                jnp.any(cv <= m_), merge,
                                lambda cm: cm, (tp, m_))
                        mk = mk & (iota16 != g) & (sg <= m_)
                        return mk, tp, m_

                    mask0 = sg <= mt
                    _mk, top, mt = lax.while_loop(
                        p1_cond, p1_body, (mask0, top, mt))
                t = jnp.max(top)

                z = jnp.zeros((16,), jnp.int32)
                a0 = z
                a1 = z
                for s in range(_SG):
                    sg = gsv[s]
                    sb = roff + s * 256
                    eb = s * 256

                    def p2_cond(carry):
                        mk, _a, _b2 = carry
                        return jnp.any(mk)

                    def p2_body(carry, sb=sb, eb=eb):
                        mk, x0, x1 = carry
                        g = jnp.max(plsc.all_reduce_ffs(mk))
                        b = sb + g * 16
                        e = eb + g * 16
                        for j in range(_GS):
                            cv = row_v[pl.ds(b + j * _Q, 16)]
                            m = cv < t
                            x0 = x0 + jnp.where(
                                m, e0_v[pl.ds(e + j * _Q, 16)], z)
                            x1 = x1 + jnp.where(
                                m, e1_v[pl.ds(e + j * _Q, 16)], z)
                        return mk & (iota16 != g), x0, x1

                    _mk, a0, a1 = lax.while_loop(
                        p2_cond, p2_body, (sg < t, a0, a1))

                s0i = jnp.sum(a0)
                s1i = jnp.sum(a1)
                sh = 4 * (iota16 & 7)
                c0 = (s0i >> sh) & 15
                c1 = (s1i >> sh) & 15
                cv16 = jnp.where(iota16 < 8, c0, c1).astype(jnp.float32)
                out_v[pl.ds((c * _CH + r) * _C, _C)] = cv16
                return 0

            lax.fori_loop(0, _CH, row_body, 0)
            return 0

        lax.fori_loop(0, _NCHUNK, chunk_body, 0)
        pltpu.sync_copy(out_v, out_hbm.at[pl.ds(base * _C, _RW * _C)])

    return body(dist_flat, gm_flat, enc0, enc1)


def _entropy_body(cnt_ref, nent_ref):
    cnt = cnt_ref[...]
    ns = jnp.sum(cnt, axis=1, keepdims=True)
    bins = cnt / ns
    nent_ref[...] = -jnp.sum(bins * jnp.log(bins + 1e-5), axis=1, keepdims=True)


def _entropy_call(counts):
    return pl.pallas_call(
        _entropy_body,
        out_shape=jax.ShapeDtypeStruct((_B, 1), jnp.float32),
    )(counts)


def kernel(encodings, categorical):
    dist, gmat = _dist_matrix(encodings, encodings.T)
    lab, enc0, enc1, gent, npop = _labels_call(categorical)
    del lab
    counts_flat = _sc_counts(
        dist.reshape(_B * _B),
        gmat.reshape(_B * _NG),
        enc0.reshape(_B),
        enc1.reshape(_B),
    )
    nent = _entropy_call(counts_flat.reshape(_B, _C))
    return (
        encodings,
        nent.reshape(_B),
        gent.reshape(()),
        npop.reshape(()),
    )


# speedup vs baseline: 2.6495x; 1.8569x over previous
import functools

import jax
import jax.numpy as jnp
from jax import lax
from jax.experimental import pallas as pl
from jax.experimental.pallas import tpu as pltpu
from jax.experimental.pallas import tpu_sc as plsc

_B = 4096
_D = 16
_C = 16
_K = 15

_NC = 2
_NS = 16
_NW = _NC * _NS
_RW = _B // _NW
_CH = 16
_NCHUNK = _RW // _CH
_GS = 4
_NG = _B // (16 * _GS)
_SG = _NG // 16
_Q = _B // _GS


def _dist_body(e_ref, et_ref, o_ref, gm_ref):
    e = e_ref[...]
    et = et_ref[...]
    x2i = jnp.sum(e * e, axis=1, keepdims=True)
    x2j = jnp.sum(et * et, axis=0, keepdims=True)
    d2 = x2i + x2j - 2.0 * jnp.dot(e, et, preferred_element_type=jnp.float32)
    d = jnp.sqrt(jnp.maximum(d2, 0.0))
    o_ref[...] = d
    q = jnp.minimum(
        jnp.minimum(d[:, 0:_Q], d[:, _Q:2 * _Q]),
        jnp.minimum(d[:, 2 * _Q:3 * _Q], d[:, 3 * _Q:4 * _Q]))
    gm_ref[...] = jnp.min(q.reshape(q.shape[0], _NG, 16), axis=2)


def _dist_matrix(encodings, encodings_t):
    rb = 256
    return pl.pallas_call(
        _dist_body,
        grid=(_B // rb,),
        in_specs=[
            pl.BlockSpec((rb, _D), lambda i: (i, 0)),
            pl.BlockSpec((_D, _B), lambda i: (0, 0)),
        ],
        out_specs=[
            pl.BlockSpec((rb, _B), lambda i: (i, 0)),
            pl.BlockSpec((rb, _NG), lambda i: (i, 0)),
        ],
        out_shape=(
            jax.ShapeDtypeStruct((_B, _B), jnp.float32),
            jax.ShapeDtypeStruct((_B, _NG), jnp.float32),
        ),
    )(encodings, encodings_t)


def _labels_body(cat_ref, lab_ref, e0_ref, e1_ref, gent_ref, npop_ref):
    cat = cat_ref[...]
    mx = jnp.max(cat, axis=1, keepdims=True)
    iota = lax.broadcasted_iota(jnp.int32, (_B, _C), 1)
    ismax = cat == mx
    lab = jnp.min(jnp.where(ismax, iota, _C), axis=1, keepdims=True)
    lab_ref[...] = lab
    one = jnp.ones_like(lab)
    sh0 = 4 * jnp.minimum(lab, 7)
    sh1 = 4 * jnp.clip(lab - 8, 0, 7)
    e0_ref[...] = jnp.where(lab < 8, one << sh0, 0)
    e1_ref[...] = jnp.where(lab >= 8, one << sh1, 0)
    onehot = (iota == lab).astype(jnp.float32)
    g = jnp.sum(onehot, axis=0)
    gb = g * jnp.float32(1.0 / _B)
    gent_ref[...] = (-jnp.sum(gb * jnp.log(gb + 1e-5)))[None, None]
    npop_ref[...] = jnp.sum((g > 0).astype(jnp.float32))[None, None]


def _labels_call(categorical):
    return pl.pallas_call(
        _labels_body,
        out_shape=(
            jax.ShapeDtypeStruct((_B, 1), jnp.int32),
            jax.ShapeDtypeStruct((_B, 1), jnp.int32),
            jax.ShapeDtypeStruct((_B, 1), jnp.int32),
            jax.ShapeDtypeStruct((1, 1), jnp.float32),
            jax.ShapeDtypeStruct((1, 1), jnp.float32),
        ),
    )(categorical)


def _sc_counts(dist_flat, gm_flat, enc0, enc1):
    mesh = plsc.VectorSubcoreMesh(core_axis_name="c", subcore_axis_name="s")

    @functools.partial(
        pl.kernel,
        mesh=mesh,
        compiler_params=pltpu.CompilerParams(needs_layout_passes=False),
        out_type=jax.ShapeDtypeStruct((_B * _C,), jnp.float32),
        scratch_types=[
            pltpu.VMEM((_CH * _B,), jnp.float32),
            pltpu.VMEM((_CH * _NG,), jnp.float32),
            pltpu.VMEM((_B,), jnp.int32),
            pltpu.VMEM((_B,), jnp.int32),
            pltpu.VMEM((_RW * _C,), jnp.float32),
        ],
    )
    def body(dist_hbm, gm_hbm, enc0_hbm, enc1_hbm, out_hbm, row_v, gm_v,
             e0_v, e1_v, out_v):
        wid = lax.axis_index("s") * _NC + lax.axis_index("c")
        base = wid * _RW
        pltpu.sync_copy(enc0_hbm, e0_v)
        pltpu.sync_copy(enc1_hbm, e1_v)
        iota16 = lax.iota(jnp.int32, 16)

        def chunk_body(c, _):
            start = base + c * _CH
            pltpu.sync_copy(dist_hbm.at[pl.ds(start * _B, _CH * _B)], row_v)
            pltpu.sync_copy(gm_hbm.at[pl.ds(start * _NG, _CH * _NG)], gm_v)

            def row_body(r, _r):
                roff = r * _B
                goff = r * _NG

                gsv = [gm_v[pl.ds(goff + s * 16, 16)] for s in range(_SG)]

                s0 = lax.sort(gsv[0])
                s1 = lax.sort(gsv[1])
                s2 = lax.sort(gsv[2])
                s3 = lax.sort(gsv[3])
                m01 = lax.sort(jnp.minimum(s0, lax.rev(s1, (0,))))
                m23 = lax.sort(jnp.minimum(s2, lax.rev(s3, (0,))))
                mm = lax.sort(jnp.minimum(m01, lax.rev(m23, (0,))))
                t0 = jnp.max(mm)

                top = jnp.full((16,), jnp.inf, jnp.float32)
                mt = t0
                for s in range(_SG):
                    sg = gsv[s]
                    sb = roff + s * 256

                    def p1_cond(carry):
                        mk, _t, _m = carry
                        return jnp.any(mk)

                    def p1_body(carry, sg=sg, sb=sb):
                        mk, tp, m_ = carry
                        g = jnp.max(plsc.all_reduce_ffs(mk))
                        b = sb + g * 16
                        for j in range(_GS):
                            cv = row_v[pl.ds(b + j * _Q, 16)]
                            tp = lax.sort(jnp.minimum(
                                tp, lax.rev(lax.sort(cv), (0,))))
                        m_ = jnp.minimum(m_, jnp.max(tp))
                        mk = mk & (iota16 != g) & (sg <= m_)
                        return mk, tp, m_

                    mask0 = sg <= mt
                    _mk, top, mt = lax.while_loop(
                        p1_cond, p1_body, (mask0, top, mt))
                t = jnp.max(top)

                z = jnp.zeros((16,), jnp.int32)
                a0 = z
                a1 = z
                for s in range(_SG):
                    sg = gsv[s]
                    sb = roff + s * 256
                    eb = s * 256

                    def p2_cond(carry):
                        mk, _a, _b2 = carry
                        return jnp.any(mk)

                    def p2_body(carry, sb=sb, eb=eb):
                        mk, x0, x1 = carry
                        g = jnp.max(plsc.all_reduce_ffs(mk))
                        b = sb + g * 16
                        e = eb + g * 16
                        for j in range(_GS):
                            cv = row_v[pl.ds(b + j * _Q, 16)]
                            m = cv < t
                            x0 = x0 + jnp.where(
                                m, e0_v[pl.ds(e + j * _Q, 16)], z)
                            x1 = x1 + jnp.where(
                                m, e1_v[pl.ds(e + j * _Q, 16)], z)
                        return mk & (iota16 != g), x0, x1

                    _mk, a0, a1 = lax.while_loop(
                        p2_cond, p2_body, (sg < t, a0, a1))

                s0i = jnp.sum(a0)
                s1i = jnp.sum(a1)
                sh = 4 * (iota16 & 7)
                c0 = (s0i >> sh) & 15
                c1 = (s1i >> sh) & 15
                cv16 = jnp.where(iota16 < 8, c0, c1).astype(jnp.float32)
                out_v[pl.ds((c * _CH + r) * _C, _C)] = cv16
                return 0

            lax.fori_loop(0, _CH, row_body, 0)
            return 0

        lax.fori_loop(0, _NCHUNK, chunk_body, 0)
        pltpu.sync_copy(out_v, out_hbm.at[pl.ds(base * _C, _RW * _C)])

    return body(dist_flat, gm_flat, enc0, enc1)


def _entropy_body(cnt_ref, nent_ref):
    cnt = cnt_ref[...]
    ns = jnp.sum(cnt, axis=1, keepdims=True)
    bins = cnt / ns
    nent_ref[...] = -jnp.sum(bins * jnp.log(bins + 1e-5), axis=1, keepdims=True)


def _entropy_call(counts):
    return pl.pallas_call(
        _entropy_body,
        out_shape=jax.ShapeDtypeStruct((_B, 1), jnp.float32),
    )(counts)


def kernel(encodings, categorical):
    dist, gmat = _dist_matrix(encodings, encodings.T)
    lab, enc0, enc1, gent, npop = _labels_call(categorical)
    del lab
    counts_flat = _sc_counts(
        dist.reshape(_B * _B),
        gmat.reshape(_B * _NG),
        enc0.reshape(_B),
        enc1.reshape(_B),
    )
    nent = _entropy_call(counts_flat.reshape(_B, _C))
    return (
        encodings,
        nent.reshape(_B),
        gent.reshape(()),
        npop.reshape(()),
    )


# speedup vs baseline: 2.7754x; 1.0475x over previous
import functools

import jax
import jax.numpy as jnp
from jax import lax
from jax.experimental import pallas as pl
from jax.experimental.pallas import tpu as pltpu
from jax.experimental.pallas import tpu_sc as plsc

_B = 4096
_D = 16
_C = 16
_K = 15

_NC = 2
_NS = 16
_NW = _NC * _NS
_RW = _B // _NW
_CH = 8
_NCHUNK = _RW // _CH
_GS = 4
_NG = _B // (16 * _GS)
_SG = _NG // 16
_Q = _B // _GS


def _dist_body(e_ref, et_ref, o_ref, gm_ref):
    e = e_ref[...]
    et = et_ref[...]
    x2i = jnp.sum(e * e, axis=1, keepdims=True)
    x2j = jnp.sum(et * et, axis=0, keepdims=True)
    d2 = x2i + x2j - 2.0 * jnp.dot(e, et, preferred_element_type=jnp.float32)
    d = jnp.sqrt(jnp.maximum(d2, 0.0))
    o_ref[...] = d
    q = jnp.minimum(
        jnp.minimum(d[:, 0:_Q], d[:, _Q:2 * _Q]),
        jnp.minimum(d[:, 2 * _Q:3 * _Q], d[:, 3 * _Q:4 * _Q]))
    gm_ref[...] = jnp.min(q.reshape(q.shape[0], _NG, 16), axis=2)


def _dist_matrix(encodings, encodings_t):
    rb = 256
    return pl.pallas_call(
        _dist_body,
        grid=(_B // rb,),
        in_specs=[
            pl.BlockSpec((rb, _D), lambda i: (i, 0)),
            pl.BlockSpec((_D, _B), lambda i: (0, 0)),
        ],
        out_specs=[
            pl.BlockSpec((rb, _B), lambda i: (i, 0)),
            pl.BlockSpec((rb, _NG), lambda i: (i, 0)),
        ],
        out_shape=(
            jax.ShapeDtypeStruct((_B, _B), jnp.float32),
            jax.ShapeDtypeStruct((_B, _NG), jnp.float32),
        ),
    )(encodings, encodings_t)


def _labels_body(cat_ref, lab_ref, e0_ref, e1_ref, gent_ref, npop_ref):
    cat = cat_ref[...]
    mx = jnp.max(cat, axis=1, keepdims=True)
    iota = lax.broadcasted_iota(jnp.int32, (_B, _C), 1)
    ismax = cat == mx
    lab = jnp.min(jnp.where(ismax, iota, _C), axis=1, keepdims=True)
    lab_ref[...] = lab
    one = jnp.ones_like(lab)
    sh0 = 4 * jnp.minimum(lab, 7)
    sh1 = 4 * jnp.clip(lab - 8, 0, 7)
    e0_ref[...] = jnp.where(lab < 8, one << sh0, 0)
    e1_ref[...] = jnp.where(lab >= 8, one << sh1, 0)
    onehot = (iota == lab).astype(jnp.float32)
    g = jnp.sum(onehot, axis=0)
    gb = g * jnp.float32(1.0 / _B)
    gent_ref[...] = (-jnp.sum(gb * jnp.log(gb + 1e-5)))[None, None]
    npop_ref[...] = jnp.sum((g > 0).astype(jnp.float32))[None, None]


def _labels_call(categorical):
    return pl.pallas_call(
        _labels_body,
        out_shape=(
            jax.ShapeDtypeStruct((_B, 1), jnp.int32),
            jax.ShapeDtypeStruct((_B, 1), jnp.int32),
            jax.ShapeDtypeStruct((_B, 1), jnp.int32),
            jax.ShapeDtypeStruct((1, 1), jnp.float32),
            jax.ShapeDtypeStruct((1, 1), jnp.float32),
        ),
    )(categorical)


def _sc_counts(dist_flat, gm_flat, enc0, enc1):
    mesh = plsc.VectorSubcoreMesh(core_axis_name="c", subcore_axis_name="s")

    @functools.partial(
        pl.kernel,
        mesh=mesh,
        compiler_params=pltpu.CompilerParams(needs_layout_passes=False),
        out_type=jax.ShapeDtypeStruct((_B * _C,), jnp.float32),
        scratch_types=[
            pltpu.VMEM((_CH * _B,), jnp.float32),
            pltpu.VMEM((_CH * _B,), jnp.float32),
            pltpu.VMEM((_CH * _NG,), jnp.float32),
            pltpu.VMEM((_CH * _NG,), jnp.float32),
            pltpu.VMEM((_B,), jnp.int32),
            pltpu.VMEM((_B,), jnp.int32),
            pltpu.VMEM((_RW * _C,), jnp.float32),
            pltpu.SemaphoreType.DMA,
            pltpu.SemaphoreType.DMA,
        ],
    )
    def body(dist_hbm, gm_hbm, enc0_hbm, enc1_hbm, out_hbm, row_a, row_b,
             gm_a, gm_b, e0_v, e1_v, out_v, sem_a, sem_b):
        wid = lax.axis_index("s") * _NC + lax.axis_index("c")
        base = wid * _RW
        pltpu.sync_copy(enc0_hbm, e0_v)
        pltpu.sync_copy(enc1_hbm, e1_v)
        iota16 = lax.iota(jnp.int32, 16)
        bufs = ((row_a, gm_a, sem_a), (row_b, gm_b, sem_b))

        def start_chunk(c, row_buf, gm_buf, sem):
            cc = jnp.minimum(c, _NCHUNK - 1)
            st = base + cc * _CH
            pltpu.async_copy(dist_hbm.at[pl.ds(st * _B, _CH * _B)], row_buf, sem)
            pltpu.async_copy(gm_hbm.at[pl.ds(st * _NG, _CH * _NG)], gm_buf, sem)

        def wait_chunk(row_buf, gm_buf, sem):
            pltpu.make_async_copy(
                dist_hbm.at[pl.ds(0, _CH * _B)], row_buf, sem).wait()
            pltpu.make_async_copy(
                gm_hbm.at[pl.ds(0, _CH * _NG)], gm_buf, sem).wait()

        start_chunk(jnp.int32(0), *bufs[0])
        start_chunk(jnp.int32(1), *bufs[1])

        def process_chunk(c, row_v, gm_v):
            def row_body(r, _r):
                roff = r * _B
                goff = r * _NG

                gsv = [gm_v[pl.ds(goff + s * 16, 16)] for s in range(_SG)]

                s0 = lax.sort(gsv[0])
                s1 = lax.sort(gsv[1])
                s2 = lax.sort(gsv[2])
                s3 = lax.sort(gsv[3])
                m01 = lax.sort(jnp.minimum(s0, lax.rev(s1, (0,))))
                m23 = lax.sort(jnp.minimum(s2, lax.rev(s3, (0,))))
                mm = lax.sort(jnp.minimum(m01, lax.rev(m23, (0,))))
                t0 = jnp.max(mm)

                top = jnp.full((16,), jnp.inf, jnp.float32)
                mt = t0
                for s in range(_SG):
                    sg = gsv[s]
                    sb = roff + s * 256

                    def p1_cond(carry):
                        mk, _t, _m = carry
                        return jnp.any(mk)

                    def p1_body(carry, sg=sg, sb=sb):
                        mk, tp, m_ = carry
                        g = jnp.max(plsc.all_reduce_ffs(mk))
                        b = sb + g * 16
                        for j in range(_GS):
                            cv = row_v[pl.ds(b + j * _Q, 16)]
                            tp = lax.sort(jnp.minimum(
                                tp, lax.rev(lax.sort(cv), (0,))))
                        m_ = jnp.minimum(m_, jnp.max(tp))
                        mk = mk & (iota16 != g) & (sg <= m_)
                        return mk, tp, m_

                    mask0 = sg <= mt
                    _mk, top, mt = lax.while_loop(
                        p1_cond, p1_body, (mask0, top, mt))
                t = jnp.max(top)

                z = jnp.zeros((16,), jnp.int32)
                a0 = z
                a1 = z
                for s in range(_SG):
                    sg = gsv[s]
                    sb = roff + s * 256
                    eb = s * 256

                    def p2_cond(carry):
                        mk, _a, _b2 = carry
                        return jnp.any(mk)

                    def p2_body(carry, sb=sb, eb=eb):
                        mk, x0, x1 = carry
                        g = jnp.max(plsc.all_reduce_ffs(mk))
                        b = sb + g * 16
                        e = eb + g * 16
                        for j in range(_GS):
                            cv = row_v[pl.ds(b + j * _Q, 16)]
                            m = cv < t
                            x0 = x0 + jnp.where(
                                m, e0_v[pl.ds(e + j * _Q, 16)], z)
                            x1 = x1 + jnp.where(
                                m, e1_v[pl.ds(e + j * _Q, 16)], z)
                        return mk & (iota16 != g), x0, x1

                    _mk, a0, a1 = lax.while_loop(
                        p2_cond, p2_body, (sg < t, a0, a1))

                s0i = jnp.sum(a0)
                s1i = jnp.sum(a1)
                sh = 4 * (iota16 & 7)
                c0 = (s0i >> sh) & 15
                c1 = (s1i >> sh) & 15
                cv16 = jnp.where(iota16 < 8, c0, c1).astype(jnp.float32)
                out_v[pl.ds((c * _CH + r) * _C, _C)] = cv16
                return 0

            lax.fori_loop(0, _CH, row_body, 0)

        def pair_body(p, _):
            for par in range(2):
                row_buf, gm_buf, sem = bufs[par]
                c = 2 * p + par
                wait_chunk(row_buf, gm_buf, sem)
                process_chunk(c, row_buf, gm_buf)
                start_chunk(c + 2, row_buf, gm_buf, sem)
            return 0

        lax.fori_loop(0, _NCHUNK // 2, pair_body, 0)
        wait_chunk(*bufs[0])
        wait_chunk(*bufs[1])
        pltpu.sync_copy(out_v, out_hbm.at[pl.ds(base * _C, _RW * _C)])

    return body(dist_flat, gm_flat, enc0, enc1)


def _entropy_body(cnt_ref, nent_ref):
    cnt = cnt_ref[...]
    ns = jnp.sum(cnt, axis=1, keepdims=True)
    bins = cnt / ns
    nent_ref[...] = -jnp.sum(bins * jnp.log(bins + 1e-5), axis=1, keepdims=True)


def _entropy_call(counts):
    return pl.pallas_call(
        _entropy_body,
        out_shape=jax.ShapeDtypeStruct((_B, 1), jnp.float32),
    )(counts)


def kernel(encodings, categorical):
    dist, gmat = _dist_matrix(encodings, encodings.T)
    lab, enc0, enc1, gent, npop = _labels_call(categorical)
    del lab
    counts_flat = _sc_counts(
        dist.reshape(_B * _B),
        gmat.reshape(_B * _NG),
        enc0.reshape(_B),
        enc1.reshape(_B),
    )
    nent = _entropy_call(counts_flat.reshape(_B, _C))
    return (
        encodings,
        nent.reshape(_B),
        gent.reshape(()),
        npop.reshape(()),
    )
